# initial kernel scaffold (unmeasured)
import jax
import jax.numpy as jnp
from jax import lax
from jax.experimental import pallas as pl
from jax.experimental.pallas import tpu as pltpu

B, S, D = 1, 1024, 2048
H, Dh, Dr = 16, 128, 32
DC = 128
SCALE = (Dh + Dr) ** -0.5
BF = jnp.bfloat16
F32 = jnp.float32


def kernel(x, Wdkv, Wuk, Wuv, Wq, Wqr, Wkr, Wo):
    def body(
        x_ref, wdkv_ref, wuk_ref, wuv_ref, wq_ref, wqr_ref, wkr_ref, wo_ref,
        out_ref,
        c_send, c_recv, wk_send, wk_recv, wv_send, wv_recv, o_buf,
        send_sems, recv_sems,
    ):
        my_x = lax.axis_index("x")
        my_y = lax.axis_index("y")
        my_z = lax.axis_index("z")
        peer = (1 - my_x, my_y, my_z)

        xb = x_ref[0].astype(BF)

        c_send[...] = jnp.dot(
            xb, wdkv_ref[...].astype(BF), preferred_element_type=F32
        ).astype(BF)
        wk_send[...] = wuk_ref[...].astype(BF)
        wv_send[...] = wuv_ref[...].astype(BF)

        barrier = pltpu.get_barrier_semaphore()
        pl.semaphore_signal(
            barrier, inc=1, device_id=peer, device_id_type=pl.DeviceIdType.MESH
        )
        pl.semaphore_wait(barrier, 1)

        rdmas = []
        pairs = [(c_send, c_recv), (wk_send, wk_recv), (wv_send, wv_recv)]
        for i, (src, dst) in enumerate(pairs):
            r = pltpu.make_async_remote_copy(
                src_ref=src,
                dst_ref=dst,
                send_sem=send_sems.at[i],
                recv_sem=recv_sems.at[i],
                device_id=peer,
                device_id_type=pl.DeviceIdType.MESH,
            )
            r.start()
            rdmas.append(r)

        Q = jnp.dot(xb, wq_ref[...].astype(BF), preferred_element_type=F32).astype(BF)
        Qr = jnp.dot(xb, wqr_ref[...].astype(BF), preferred_element_type=F32).astype(BF)
        Kr = jnp.dot(xb, wkr_ref[...].astype(BF), preferred_element_type=F32).astype(BF)
        c_loc = c_send[...]
        Kp = jnp.dot(c_loc, wk_send[...], preferred_element_type=F32)
        Vp = jnp.dot(c_loc, wv_send[...], preferred_element_type=F32)

        for r in rdmas:
            r.wait()

        K = (Kp + jnp.dot(c_recv[...], wk_recv[...], preferred_element_type=F32)).astype(BF)
        V = (Vp + jnp.dot(c_recv[...], wv_recv[...], preferred_element_type=F32)).astype(BF)

        contract = (((1,), (1,)), ((), ()))
        kr = Kr
        for h in range(H):
            q = Q[:, h * Dh:(h + 1) * Dh]
            k = K[:, h * Dh:(h + 1) * Dh]
            v = V[:, h * Dh:(h + 1) * Dh]
            qr = Qr[:, h * Dr:(h + 1) * Dr]
            s = lax.dot_general(q, k, contract, preferred_element_type=F32)
            s = s + lax.dot_general(qr, kr, contract, preferred_element_type=F32)
            s = s * SCALE
            m = jnp.max(s, axis=-1, keepdims=True)
            p = jnp.exp(s - m)
            denom = jnp.sum(p, axis=-1, keepdims=True)
            prob = (p / denom).astype(BF)
            o_buf[:, h * Dh:(h + 1) * Dh] = jnp.dot(
                prob, v, preferred_element_type=F32
            ).astype(BF)

        out_ref[0] = jnp.dot(
            o_buf[...], wo_ref[...].astype(BF), preferred_element_type=F32
        )

    return pl.pallas_call(
        body,
        out_shape=jax.ShapeDtypeStruct((B, S, D), F32),
        in_specs=[pl.BlockSpec(memory_space=pltpu.VMEM)] * 8,
        out_specs=pl.BlockSpec(memory_space=pltpu.VMEM),
        scratch_shapes=[
            pltpu.VMEM((S, DC), BF),
            pltpu.VMEM((S, DC), BF),
            pltpu.VMEM((DC, D), BF),
            pltpu.VMEM((DC, D), BF),
            pltpu.VMEM((DC, D), BF),
            pltpu.VMEM((DC, D), BF),
            pltpu.VMEM((S, D), BF),
            pltpu.SemaphoreType.DMA((3,)),
            pltpu.SemaphoreType.DMA((3,)),
        ],
        compiler_params=pltpu.CompilerParams(collective_id=0),
    )(x, Wdkv, Wuk, Wuv, Wq, Wqr, Wkr, Wo)


# baseline (device time: 144785 ns/iter reference)
import jax
import jax.numpy as jnp
from jax import lax
from jax.experimental import pallas as pl
from jax.experimental.pallas import tpu as pltpu

B, S, D = 1, 1024, 2048
H, Dh, Dr = 16, 128, 32
DC = 128
SCALE = (Dh + Dr) ** -0.5
BF = jnp.bfloat16
F32 = jnp.float32


def kernel(x, Wdkv, Wuk, Wuv, Wq, Wqr, Wkr, Wo):
    x, Wdkv, Wuk, Wuv, Wq, Wqr, Wkr, Wo = (
        a.astype(BF) for a in (x, Wdkv, Wuk, Wuv, Wq, Wqr, Wkr, Wo)
    )

    def body(
        x_ref, wdkv_ref, wuk_ref, wuv_ref, wq_ref, wqr_ref, wkr_ref, wo_ref,
        out_ref,
        c_send, c_recv, wk_recv, wv_recv, q_buf, k_buf, v_buf,
        send_sems, recv_sems,
    ):
        my_x = lax.axis_index("x")
        my_y = lax.axis_index("y")
        my_z = lax.axis_index("z")
        peer = (1 - my_x, my_y, my_z)

        xb = x_ref[0]

        c_send[...] = jnp.dot(
            xb, wdkv_ref[...], preferred_element_type=F32
        ).astype(BF)

        barrier = pltpu.get_barrier_semaphore()
        pl.semaphore_signal(
            barrier, inc=1, device_id=peer, device_id_type=pl.DeviceIdType.MESH
        )
        pl.semaphore_wait(barrier, 1)

        rdmas = []
        pairs = [(c_send, c_recv), (wuk_ref, wk_recv), (wuv_ref, wv_recv)]
        for i, (src, dst) in enumerate(pairs):
            r = pltpu.make_async_remote_copy(
                src_ref=src,
                dst_ref=dst,
                send_sem=send_sems.at[i],
                recv_sem=recv_sems.at[i],
                device_id=peer,
                device_id_type=pl.DeviceIdType.MESH,
            )
            r.start()
            rdmas.append(r)

        q_buf[...] = jnp.dot(xb, wq_ref[...], preferred_element_type=F32).astype(BF)
        Qr = jnp.dot(xb, wqr_ref[...], preferred_element_type=F32).astype(BF)
        Kr = jnp.dot(xb, wkr_ref[...], preferred_element_type=F32).astype(BF)
        c_loc = c_send[...]
        k_buf[...] = jnp.dot(c_loc, wuk_ref[...], preferred_element_type=F32).astype(BF)
        v_buf[...] = jnp.dot(c_loc, wuv_ref[...], preferred_element_type=F32).astype(BF)

        for r in rdmas:
            r.wait()

        k_buf[...] += jnp.dot(
            c_recv[...], wk_recv[...], preferred_element_type=F32
        ).astype(BF)
        v_buf[...] += jnp.dot(
            c_recv[...], wv_recv[...], preferred_element_type=F32
        ).astype(BF)

        contract = (((1,), (1,)), ((), ()))
        for h in range(H):
            q = q_buf[:, h * Dh:(h + 1) * Dh]
            k = k_buf[:, h * Dh:(h + 1) * Dh]
            v = v_buf[:, h * Dh:(h + 1) * Dh]
            qr = Qr[:, h * Dr:(h + 1) * Dr]
            s = lax.dot_general(q, k, contract, preferred_element_type=F32)
            s = s + lax.dot_general(qr, Kr, contract, preferred_element_type=F32)
            s = s * SCALE
            m = jnp.max(s, axis=-1, keepdims=True)
            p = jnp.exp(s - m)
            denom = jnp.sum(p, axis=-1, keepdims=True)
            prob = (p / denom).astype(BF)
            o_h = jnp.dot(prob, v, preferred_element_type=F32).astype(BF)
            contrib = jnp.dot(
                o_h, wo_ref[h * Dh:(h + 1) * Dh, :], preferred_element_type=F32
            )
            if h == 0:
                out_ref[0] = contrib
            else:
                out_ref[0] += contrib

    return pl.pallas_call(
        body,
        out_shape=jax.ShapeDtypeStruct((B, S, D), F32),
        in_specs=[pl.BlockSpec(memory_space=pltpu.VMEM)] * 8,
        out_specs=pl.BlockSpec(memory_space=pltpu.VMEM),
        scratch_shapes=[
            pltpu.VMEM((S, DC), BF),
            pltpu.VMEM((S, DC), BF),
            pltpu.VMEM((DC, D), BF),
            pltpu.VMEM((DC, D), BF),
            pltpu.VMEM((S, D), BF),
            pltpu.VMEM((S, D), BF),
            pltpu.VMEM((S, D), BF),
            pltpu.SemaphoreType.DMA((3,)),
            pltpu.SemaphoreType.DMA((3,)),
        ],
        compiler_params=pltpu.CompilerParams(
            collective_id=0,
            vmem_limit_bytes=100 * 1024 * 1024,
        ),
    )(x, Wdkv, Wuk, Wuv, Wq, Wqr, Wkr, Wo)


# device time: 125337 ns/iter; 1.1552x vs baseline; 1.1552x over previous
import jax
import jax.numpy as jnp
from jax import lax
from jax.experimental import pallas as pl
from jax.experimental.pallas import tpu as pltpu

B, S, D = 1, 1024, 2048
H, Dh, Dr = 16, 128, 32
DC = 128
SCALE = (Dh + Dr) ** -0.5
BF = jnp.bfloat16
F32 = jnp.float32


def kernel(x, Wdkv, Wuk, Wuv, Wq, Wqr, Wkr, Wo):
    x, Wdkv, Wuk, Wuv, Wq, Wqr, Wkr, Wo = (
        a.astype(BF) for a in (x, Wdkv, Wuk, Wuv, Wq, Wqr, Wkr, Wo)
    )

    def body(
        x_ref, wdkv_ref, wuk_ref, wuv_ref, wq_ref, wqr_ref, wkr_ref, wo_ref,
        out_ref,
        c_send, c_recv, wk_recv, wv_recv, q_buf, k_buf, v_buf,
        send_sems, recv_sems,
    ):
        my_x = lax.axis_index("x")
        my_y = lax.axis_index("y")
        my_z = lax.axis_index("z")
        peer = (1 - my_x, my_y, my_z)

        xb = x_ref[0]

        c_send[...] = jnp.dot(
            xb, wdkv_ref[...], preferred_element_type=F32
        ).astype(BF)

        barrier = pltpu.get_barrier_semaphore()
        pl.semaphore_signal(
            barrier, inc=1, device_id=peer, device_id_type=pl.DeviceIdType.MESH
        )
        pl.semaphore_wait(barrier, 1)

        rdmas = []
        pairs = [(c_send, c_recv), (wuk_ref, wk_recv), (wuv_ref, wv_recv)]
        for i, (src, dst) in enumerate(pairs):
            r = pltpu.make_async_remote_copy(
                src_ref=src,
                dst_ref=dst,
                send_sem=send_sems.at[i],
                recv_sem=recv_sems.at[i],
                device_id=peer,
                device_id_type=pl.DeviceIdType.MESH,
            )
            r.start()
            rdmas.append(r)

        q_buf[...] = jnp.dot(xb, wq_ref[...], preferred_element_type=F32).astype(BF)
        Qr = jnp.dot(xb, wqr_ref[...], preferred_element_type=F32).astype(BF)
        Kr = jnp.dot(xb, wkr_ref[...], preferred_element_type=F32).astype(BF)
        c_loc = c_send[...]
        k_buf[...] = jnp.dot(c_loc, wuk_ref[...], preferred_element_type=F32).astype(BF)
        v_buf[...] = jnp.dot(c_loc, wuv_ref[...], preferred_element_type=F32).astype(BF)

        for r in rdmas:
            r.wait()

        k_buf[...] += jnp.dot(
            c_recv[...], wk_recv[...], preferred_element_type=F32
        ).astype(BF)
        v_buf[...] += jnp.dot(
            c_recv[...], wv_recv[...], preferred_element_type=F32
        ).astype(BF)

        contract = (((1,), (1,)), ((), ()))
        for h in range(H):
            q = q_buf[:, h * Dh:(h + 1) * Dh]
            k = k_buf[:, h * Dh:(h + 1) * Dh]
            v = v_buf[:, h * Dh:(h + 1) * Dh]
            qr = Qr[:, h * Dr:(h + 1) * Dr]
            s = lax.dot_general(q, k, contract, preferred_element_type=F32)
            s = s + lax.dot_general(qr, Kr, contract, preferred_element_type=F32)
            p = jnp.exp(s * SCALE)
            denom = jnp.sum(p, axis=-1, keepdims=True)
            o_h = jnp.dot(p.astype(BF), v, preferred_element_type=F32)
            o_h = (o_h / denom).astype(BF)
            contrib = jnp.dot(
                o_h, wo_ref[h * Dh:(h + 1) * Dh, :], preferred_element_type=F32
            )
            if h == 0:
                out_ref[0] = contrib
            else:
                out_ref[0] += contrib

    return pl.pallas_call(
        body,
        out_shape=jax.ShapeDtypeStruct((B, S, D), F32),
        in_specs=[pl.BlockSpec(memory_space=pltpu.VMEM)] * 8,
        out_specs=pl.BlockSpec(memory_space=pltpu.VMEM),
        scratch_shapes=[
            pltpu.VMEM((S, DC), BF),
            pltpu.VMEM((S, DC), BF),
            pltpu.VMEM((DC, D), BF),
            pltpu.VMEM((DC, D), BF),
            pltpu.VMEM((S, D), BF),
            pltpu.VMEM((S, D), BF),
            pltpu.VMEM((S, D), BF),
            pltpu.SemaphoreType.DMA((3,)),
            pltpu.SemaphoreType.DMA((3,)),
        ],
        compiler_params=pltpu.CompilerParams(
            collective_id=0,
            vmem_limit_bytes=100 * 1024 * 1024,
        ),
    )(x, Wdkv, Wuk, Wuv, Wq, Wqr, Wkr, Wo)
